# baseline (speedup 1.0000x reference)
import functools

import jax
import jax.numpy as jnp
from jax import lax
from jax.experimental import pallas as pl
from jax.experimental.pallas import tpu as pltpu, tpu_sc as plsc

NUM_CHAMPIONS = 100000
BATCH = 16384
TEAM = 5
NUM_WORKERS = 32
ROWS_PER_WORKER = BATCH // NUM_WORKERS
FLAT_PER_WORKER = ROWS_PER_WORKER * TEAM
GCHUNK = 128
NCHUNKS = FLAT_PER_WORKER // GCHUNK
LANES = 16
CHUNKS = ROWS_PER_WORKER // LANES


@functools.partial(
    pl.kernel,
    mesh=plsc.VectorSubcoreMesh(core_axis_name="c", subcore_axis_name="s"),
    out_type=jax.ShapeDtypeStruct((BATCH,), jnp.float32),
    compiler_params=pltpu.CompilerParams(needs_layout_passes=False),
    scratch_types=[
        pltpu.VMEM((NCHUNKS, GCHUNK), jnp.int32),
        pltpu.VMEM((NCHUNKS, GCHUNK), jnp.int32),
        pltpu.VMEM((FLAT_PER_WORKER,), jnp.float32),
        pltpu.VMEM((FLAT_PER_WORKER,), jnp.float32),
        pltpu.VMEM((ROWS_PER_WORKER,), jnp.float32),
        pltpu.SemaphoreType.DMA,
    ],
)
def _pam_kernel(table_hbm, red_hbm, blue_hbm, out_hbm,
                ridx_v, bidx_v, rvals_v, bvals_v, out_v, sem):
    wid = lax.axis_index("s") * 2 + lax.axis_index("c")
    base = wid * ROWS_PER_WORKER

    pltpu.sync_copy(red_hbm.at[wid], ridx_v)
    pltpu.sync_copy(blue_hbm.at[wid], bidx_v)

    handles = []
    for j in range(NCHUNKS):
        sl = pl.ds(j * GCHUNK, GCHUNK)
        handles.append(pltpu.async_copy(table_hbm.at[ridx_v.at[j]], rvals_v.at[sl], sem))
        handles.append(pltpu.async_copy(table_hbm.at[bidx_v.at[j]], bvals_v.at[sl], sem))
    for h in handles:
        h.wait()

    lanes5 = lax.iota(jnp.int32, LANES) * TEAM
    for i in range(CHUNKS):
        acc = plsc.load_gather(rvals_v, [lanes5 + (i * LANES * TEAM)])
        for t in range(1, TEAM):
            acc = acc + plsc.load_gather(rvals_v, [lanes5 + (i * LANES * TEAM + t)])
        for t in range(TEAM):
            acc = acc - plsc.load_gather(bvals_v, [lanes5 + (i * LANES * TEAM + t)])
        out_v[pl.ds(i * LANES, LANES)] = 1.0 / (1.0 + jnp.exp(-acc))

    pltpu.sync_copy(out_v, out_hbm.at[pl.ds(base, ROWS_PER_WORKER)])


def kernel(red, blue, strengths):
    red2 = red.reshape(NUM_WORKERS, NCHUNKS, GCHUNK)
    blue2 = blue.reshape(NUM_WORKERS, NCHUNKS, GCHUNK)
    return _pam_kernel(strengths, red2, blue2)


# speedup vs baseline: 1.9120x; 1.9120x over previous
import functools

import jax
import jax.numpy as jnp
from jax import lax
from jax.experimental import pallas as pl
from jax.experimental.pallas import tpu as pltpu, tpu_sc as plsc

NUM_CHAMPIONS = 100000
BATCH = 16384
TEAM = 5
NUM_WORKERS = 32
ROWS_PER_WORKER = BATCH // NUM_WORKERS
FLAT_PER_WORKER = ROWS_PER_WORKER * 2 * TEAM
GCHUNK = 128
NCHUNKS = FLAT_PER_WORKER // GCHUNK
LANES = 16
CHUNKS = ROWS_PER_WORKER // LANES


@functools.partial(
    pl.kernel,
    mesh=plsc.VectorSubcoreMesh(core_axis_name="c", subcore_axis_name="s"),
    out_type=jax.ShapeDtypeStruct((BATCH,), jnp.float32),
    compiler_params=pltpu.CompilerParams(needs_layout_passes=False),
    scratch_types=[
        pltpu.VMEM((NCHUNKS, GCHUNK), jnp.int32),
        pltpu.VMEM((FLAT_PER_WORKER,), jnp.float32),
        pltpu.VMEM((ROWS_PER_WORKER,), jnp.float32),
        pltpu.SemaphoreType.DMA,
    ],
)
def _pam_kernel(table_hbm, idx_hbm, out_hbm, idx_v, vals_v, out_v, sem):
    wid = lax.axis_index("s") * 2 + lax.axis_index("c")
    base = wid * ROWS_PER_WORKER

    pltpu.sync_copy(idx_hbm.at[wid], idx_v)

    handles = []
    for j in range(NCHUNKS):
        handles.append(pltpu.async_copy(
            table_hbm.at[idx_v.at[j]], vals_v.at[pl.ds(j * GCHUNK, GCHUNK)], sem))
    for h in handles:
        h.wait()

    for i in range(CHUNKS):
        sl0 = pl.ds(i * LANES, LANES)
        acc = vals_v[sl0]
        for t in range(1, TEAM):
            acc = acc + vals_v[pl.ds(t * ROWS_PER_WORKER + i * LANES, LANES)]
        for t in range(TEAM, 2 * TEAM):
            acc = acc - vals_v[pl.ds(t * ROWS_PER_WORKER + i * LANES, LANES)]
        out_v[sl0] = 1.0 / (1.0 + jnp.exp(-acc))

    pltpu.sync_copy(out_v, out_hbm.at[pl.ds(base, ROWS_PER_WORKER)])


def kernel(red, blue, strengths):
    idx = jnp.concatenate([red.T, blue.T], axis=0).astype(jnp.int32)
    idx = idx.reshape(2 * TEAM, NUM_WORKERS, ROWS_PER_WORKER).transpose(1, 0, 2)
    idx = idx.reshape(NUM_WORKERS, NCHUNKS, GCHUNK)
    return _pam_kernel(strengths, idx)
